# Initial kernel scaffold; baseline (speedup 1.0000x reference)
import jax
import jax.numpy as jnp
from jax.experimental import pallas as pl


def kernel(x, adj, W1, b1, W2, b2):
    raise NotImplementedError("write your pallas kernel here")


# speedup vs baseline: 6.9428x; 6.9428x over previous
import functools

import jax
import jax.numpy as jnp
from jax import lax
from jax.experimental import pallas as pl
from jax.experimental.pallas import tpu as pltpu
from jax.experimental.pallas import tpu_sc as plsc

N = 10000
E = 320000
D_IN = 128
HID = 256
HALF = 128
NCLS = 64

C = 128
NT = 16
NSC = 2
NPAD = 10240
ROWS_PT = NPAD // NT
DEGW = 128
E_PAD = 327680
BUF_CH = 80

_mesh = plsc.VectorSubcoreMesh(core_axis_name="c", subcore_axis_name="s")


_DEG_NCH = E_PAD // (NSC * NT * C)


@functools.partial(
    pl.kernel,
    out_type=jax.ShapeDtypeStruct((NSC * NPAD, DEGW), jnp.float32),
    mesh=_mesh,
    scratch_types=[
        pltpu.VMEM_SHARED((NPAD, DEGW), jnp.float32),
        pltpu.VMEM((_DEG_NCH, C), jnp.int32),
        pltpu.VMEM((C, DEGW), jnp.float32),
    ],
)
def _deg_kernel(dst3d, ones, zeros, deg_out, deg_sh, dstb, ones_v):
    cid = lax.axis_index("c")
    sid = lax.axis_index("s")
    wid = cid * NT + sid
    rs = pl.ds(sid * ROWS_PT, ROWS_PT)
    pltpu.sync_copy(zeros, deg_sh.at[rs])
    pltpu.sync_copy(ones, ones_v)
    pltpu.sync_copy(dst3d.at[wid], dstb)
    plsc.subcore_barrier()

    def body(j, carry):
        pltpu.sync_copy(ones_v, deg_sh.at[dstb.at[j]], add=True)
        return carry

    lax.fori_loop(0, _DEG_NCH, body, 0)
    plsc.subcore_barrier()
    pltpu.sync_copy(deg_sh.at[rs],
                    deg_out.at[pl.ds(cid * NPAD + sid * ROWS_PT, ROWS_PT)])


def _make_scatter(width, split_edges):
    nch = E_PAD // (NT * C) // (NSC if split_edges else 1)
    rounds = nch // BUF_CH if nch > BUF_CH else 1
    buf_ch = nch // rounds

    @functools.partial(
        pl.kernel,
        out_type=[jax.ShapeDtypeStruct((NPAD, width), jnp.float32),
                  jax.ShapeDtypeStruct((NPAD, width), jnp.float32)],
        mesh=_mesh,
        scratch_types=[
            pltpu.VMEM_SHARED((NPAD, width), jnp.float32),
            pltpu.VMEM((buf_ch, C), jnp.int32),
            pltpu.VMEM((buf_ch, C), jnp.int32),
            pltpu.VMEM((C, width), jnp.float32),
            pltpu.SemaphoreType.DMA,
        ],
    )
    def k(src3d, dst3d, tabA, tabB, zeros, outA, outB,
          acc_sh, srcb, dstb, rows_v, sem):
        cid = lax.axis_index("c")
        sid = lax.axis_index("s")
        if split_edges:
            tile0 = cid * NT + sid
        else:
            tile0 = sid
        rs = pl.ds(sid * ROWS_PT, ROWS_PT)
        pltpu.sync_copy(zeros, acc_sh.at[rs])
        plsc.subcore_barrier()

        def run(tab):
            for r in range(rounds):
                pltpu.sync_copy(src3d.at[tile0, pl.ds(r * buf_ch, buf_ch)], srcb)
                pltpu.sync_copy(dst3d.at[tile0, pl.ds(r * buf_ch, buf_ch)], dstb)

                def body(j, carry):
                    pltpu.async_copy(tab.at[srcb.at[j]], rows_v, sem).wait()
                    pltpu.sync_copy(rows_v, acc_sh.at[dstb.at[j]], add=True)
                    return carry

                lax.fori_loop(0, buf_ch, body, 0)

        @pl.when(cid == 0)
        def _():
            run(tabA)

        @pl.when(cid == 1)
        def _():
            run(tabB)

        plsc.subcore_barrier()

        @pl.when(cid == 0)
        def _():
            pltpu.sync_copy(acc_sh.at[rs], outA.at[rs])

        @pl.when(cid == 1)
        def _():
            pltpu.sync_copy(acc_sh.at[rs], outB.at[rs])

    return k


_scatter_h = _make_scatter(HALF, split_edges=False)
_scatter_o = _make_scatter(HALF, split_edges=True)


def _isd_from(dA, dB):
    deg = dA[...][:, 0] + dB[...][:, 0]
    return lax.rsqrt(jnp.maximum(deg, 1.0))


def _mm1_body(x_ref, w_ref, dA, dB, lo_ref, hi_ref):
    isd = _isd_from(dA, dB)
    s = jnp.dot(x_ref[...] * isd[:, None], w_ref[...],
                preferred_element_type=jnp.float32,
                precision=lax.Precision.HIGHEST)
    lo_ref[...] = s[:, :HALF]
    hi_ref[...] = s[:, HALF:]


def _mm2_body(lo_ref, hi_ref, dA, dB, b1_ref, w2_ref, out_ref):
    isd = _isd_from(dA, dB)[:, None]
    b = b1_ref[...]
    h_lo = jnp.maximum(lo_ref[...] * isd + b[:, :HALF], 0.0) * isd
    h_hi = jnp.maximum(hi_ref[...] * isd + b[:, HALF:], 0.0) * isd
    w2 = w2_ref[...]
    out_ref[...] = (
        jnp.dot(h_lo, w2[:HALF], preferred_element_type=jnp.float32,
                precision=lax.Precision.HIGHEST)
        + jnp.dot(h_hi, w2[HALF:], preferred_element_type=jnp.float32,
                  precision=lax.Precision.HIGHEST))


def _out_body(aA, aB, dA, dB, b2_ref, o_ref):
    isd = _isd_from(dA, dB)[:, None]
    o = (aA[...] + aB[...])[:, :NCLS] * isd + b2_ref[...]
    m = jnp.max(o, axis=1, keepdims=True)
    y = o - m
    o_ref[...] = y - jnp.log(jnp.sum(jnp.exp(y), axis=1, keepdims=True))


_R = 2000


def _row_spec(w):
    return pl.BlockSpec((_R, w), lambda i: (i, 0))


def _full_spec(a, b):
    return pl.BlockSpec((a, b), lambda i: (0, 0))


def kernel(x, adj, W1, b1, W2, b2):
    src = adj[0].astype(jnp.int32)
    dst = adj[1].astype(jnp.int32)
    pad = jnp.full((E_PAD - E,), N, jnp.int32)
    src_p = jnp.concatenate([src, pad])
    dst_p = jnp.concatenate([dst, pad])
    src3d_16 = src_p.reshape(NT, E_PAD // (NT * C), C)
    dst3d_16 = dst_p.reshape(NT, E_PAD // (NT * C), C)
    src3d_32 = src_p.reshape(NSC * NT, _DEG_NCH, C)
    dst3d_32 = dst_p.reshape(NSC * NT, _DEG_NCH, C)
    ones128 = jnp.ones((C, DEGW), jnp.float32)
    zer128 = jnp.zeros((ROWS_PT, HALF), jnp.float32)

    deg2 = _deg_kernel(dst3d_32, ones128, zer128).reshape(NSC, NPAD, DEGW)
    degA, degB = deg2[0], deg2[1]

    grid = (N // _R,)
    s_lo, s_hi = pl.pallas_call(
        _mm1_body,
        grid=grid,
        in_specs=[_row_spec(D_IN), _full_spec(D_IN, HID),
                  _row_spec(DEGW), _row_spec(DEGW)],
        out_specs=[_row_spec(HALF), _row_spec(HALF)],
        out_shape=[jax.ShapeDtypeStruct((NPAD, HALF), jnp.float32)] * 2,
    )(x, W1, degA, degB)

    acc_lo, acc_hi = _scatter_h(src3d_16, dst3d_16, s_lo, s_hi, zer128)

    W2p = jnp.pad(W2, ((0, 0), (0, HALF - NCLS)))
    s2 = pl.pallas_call(
        _mm2_body,
        grid=grid,
        in_specs=[_row_spec(HALF), _row_spec(HALF),
                  _row_spec(DEGW), _row_spec(DEGW),
                  _full_spec(1, HID), _full_spec(HID, HALF)],
        out_specs=_row_spec(HALF),
        out_shape=jax.ShapeDtypeStruct((NPAD, HALF), jnp.float32),
    )(acc_lo, acc_hi, degA, degB, b1.reshape(1, HID), W2p)

    acc2A, acc2B = _scatter_o(src3d_32, dst3d_32, s2, s2, zer128)

    out = pl.pallas_call(
        _out_body,
        grid=grid,
        in_specs=[_row_spec(HALF), _row_spec(HALF),
                  _row_spec(DEGW), _row_spec(DEGW),
                  _full_spec(1, NCLS)],
        out_specs=_row_spec(NCLS),
        out_shape=jax.ShapeDtypeStruct((N, NCLS), jnp.float32),
    )(acc2A, acc2B, degA, degB, b2.reshape(1, NCLS))
    return out


# speedup vs baseline: 18.2192x; 2.6242x over previous
import functools

import jax
import jax.numpy as jnp
from jax import lax
from jax.experimental import pallas as pl
from jax.experimental.pallas import tpu as pltpu
from jax.experimental.pallas import tpu_sc as plsc

N = 10000
E = 320000
D_IN = 128
HID = 256
HALF = 128
NCLS = 64

C = 128
NT = 16
NSC = 2
NPAD = 10240
ROWS_PT = NPAD // NT
DEGW = 128
E_PAD = 327680
BUF_CH = 40

_mesh = plsc.VectorSubcoreMesh(core_axis_name="c", subcore_axis_name="s")


_DEG_NCH = E_PAD // (NSC * NT * C)


@functools.partial(
    pl.kernel,
    out_type=jax.ShapeDtypeStruct((NSC * NPAD, DEGW), jnp.float32),
    mesh=_mesh,
    scratch_types=[
        pltpu.VMEM_SHARED((NPAD, DEGW), jnp.float32),
        pltpu.VMEM((_DEG_NCH, C), jnp.int32),
        pltpu.VMEM((C, DEGW), jnp.float32),
    ],
)
def _deg_kernel(dst3d, ones, zeros, deg_out, deg_sh, dstb, ones_v):
    cid = lax.axis_index("c")
    sid = lax.axis_index("s")
    wid = cid * NT + sid
    rs = pl.ds(sid * ROWS_PT, ROWS_PT)
    pltpu.sync_copy(zeros, deg_sh.at[rs])
    pltpu.sync_copy(ones, ones_v)
    pltpu.sync_copy(dst3d.at[wid], dstb)
    plsc.subcore_barrier()

    def body(j, carry):
        pltpu.sync_copy(ones_v, deg_sh.at[dstb.at[j]], add=True)
        return carry

    lax.fori_loop(0, _DEG_NCH, body, 0)
    plsc.subcore_barrier()
    pltpu.sync_copy(deg_sh.at[rs],
                    deg_out.at[pl.ds(cid * NPAD + sid * ROWS_PT, ROWS_PT)])


def _make_scatter(width, split_edges):
    nch = E_PAD // (NT * C) // (NSC if split_edges else 1)
    rounds = nch // BUF_CH if nch > BUF_CH else 1
    buf_ch = nch // rounds
    npairs = buf_ch // 2

    @functools.partial(
        pl.kernel,
        out_type=[jax.ShapeDtypeStruct((NPAD, width), jnp.float32),
                  jax.ShapeDtypeStruct((NPAD, width), jnp.float32)],
        mesh=_mesh,
        scratch_types=[
            pltpu.VMEM_SHARED((NPAD, width), jnp.float32),
            pltpu.VMEM((buf_ch, C), jnp.int32),
            pltpu.VMEM((buf_ch, C), jnp.int32),
            pltpu.VMEM((C, width), jnp.float32),
            pltpu.VMEM((C, width), jnp.float32),
            pltpu.SemaphoreType.DMA,
            pltpu.SemaphoreType.DMA,
        ],
    )
    def k(src3d, dst3d, tabA, tabB, zeros, outA, outB,
          acc_sh, srcb, dstb, rv0, rv1, sem0, sem1):
        cid = lax.axis_index("c")
        sid = lax.axis_index("s")
        if split_edges:
            tile0 = cid * NT + sid
        else:
            tile0 = sid
        rs = pl.ds(sid * ROWS_PT, ROWS_PT)
        pltpu.sync_copy(zeros, acc_sh.at[rs])
        plsc.subcore_barrier()

        def run(tab):
            for r in range(rounds):
                pltpu.sync_copy(src3d.at[tile0, pl.ds(r * buf_ch, buf_ch)], srcb)
                pltpu.sync_copy(dst3d.at[tile0, pl.ds(r * buf_ch, buf_ch)], dstb)
                pltpu.async_copy(tab.at[srcb.at[0]], rv0, sem0)

                def body(k_, carry):
                    j0 = 2 * k_
                    pltpu.async_copy(tab.at[srcb.at[j0 + 1]], rv1, sem1)
                    pltpu.make_async_copy(tab.at[srcb.at[j0]], rv0, sem0).wait()
                    pltpu.sync_copy(rv0, acc_sh.at[dstb.at[j0]], add=True)

                    @pl.when(k_ < npairs - 1)
                    def _():
                        pltpu.async_copy(tab.at[srcb.at[j0 + 2]], rv0, sem0)

                    pltpu.make_async_copy(tab.at[srcb.at[j0 + 1]], rv1, sem1).wait()
                    pltpu.sync_copy(rv1, acc_sh.at[dstb.at[j0 + 1]], add=True)
                    return carry

                lax.fori_loop(0, npairs, body, 0)

        @pl.when(cid == 0)
        def _():
            run(tabA)

        @pl.when(cid == 1)
        def _():
            run(tabB)

        plsc.subcore_barrier()

        @pl.when(cid == 0)
        def _():
            pltpu.sync_copy(acc_sh.at[rs], outA.at[rs])

        @pl.when(cid == 1)
        def _():
            pltpu.sync_copy(acc_sh.at[rs], outB.at[rs])

    return k


_scatter_h = _make_scatter(HALF, split_edges=False)
_scatter_o = _make_scatter(HALF, split_edges=True)


def _isd_from(dA, dB):
    deg = dA[...][:, 0] + dB[...][:, 0]
    return lax.rsqrt(jnp.maximum(deg, 1.0))


def _mm1_body(x_ref, w_ref, dA, dB, lo_ref, hi_ref):
    isd = _isd_from(dA, dB)
    s = jnp.dot(x_ref[...] * isd[:, None], w_ref[...],
                preferred_element_type=jnp.float32,
                precision=lax.Precision.HIGHEST)
    lo_ref[...] = s[:, :HALF]
    hi_ref[...] = s[:, HALF:]


def _mm2_body(lo_ref, hi_ref, dA, dB, b1_ref, w2_ref, out_ref):
    isd = _isd_from(dA, dB)[:, None]
    b = b1_ref[...]
    h_lo = jnp.maximum(lo_ref[...] * isd + b[:, :HALF], 0.0) * isd
    h_hi = jnp.maximum(hi_ref[...] * isd + b[:, HALF:], 0.0) * isd
    w2 = w2_ref[...]
    out_ref[...] = (
        jnp.dot(h_lo, w2[:HALF], preferred_element_type=jnp.float32,
                precision=lax.Precision.HIGHEST)
        + jnp.dot(h_hi, w2[HALF:], preferred_element_type=jnp.float32,
                  precision=lax.Precision.HIGHEST))


def _out_body(aA, aB, dA, dB, b2_ref, o_ref):
    isd = _isd_from(dA, dB)[:, None]
    o = (aA[...] + aB[...])[:, :NCLS] * isd + b2_ref[...]
    m = jnp.max(o, axis=1, keepdims=True)
    y = o - m
    o_ref[...] = y - jnp.log(jnp.sum(jnp.exp(y), axis=1, keepdims=True))


_R = 2000


def _row_spec(w):
    return pl.BlockSpec((_R, w), lambda i: (i, 0))


def _full_spec(a, b):
    return pl.BlockSpec((a, b), lambda i: (0, 0))


def kernel(x, adj, W1, b1, W2, b2):
    src = adj[0].astype(jnp.int32)
    dst = adj[1].astype(jnp.int32)
    pad = N + jnp.arange(E_PAD - E, dtype=jnp.int32) % (NPAD - N)
    src_p = jnp.concatenate([src, pad])
    dst_p = jnp.concatenate([dst, pad])
    src3d_16 = src_p.reshape(NT, E_PAD // (NT * C), C)
    dst3d_16 = dst_p.reshape(NT, E_PAD // (NT * C), C)
    src3d_32 = src_p.reshape(NSC * NT, _DEG_NCH, C)
    dst3d_32 = dst_p.reshape(NSC * NT, _DEG_NCH, C)
    ones128 = jnp.ones((C, DEGW), jnp.float32)
    zer128 = jnp.zeros((ROWS_PT, HALF), jnp.float32)

    deg2 = _deg_kernel(dst3d_32, ones128, zer128).reshape(NSC, NPAD, DEGW)
    degA, degB = deg2[0], deg2[1]

    grid = (N // _R,)
    s_lo, s_hi = pl.pallas_call(
        _mm1_body,
        grid=grid,
        in_specs=[_row_spec(D_IN), _full_spec(D_IN, HID),
                  _row_spec(DEGW), _row_spec(DEGW)],
        out_specs=[_row_spec(HALF), _row_spec(HALF)],
        out_shape=[jax.ShapeDtypeStruct((NPAD, HALF), jnp.float32)] * 2,
    )(x, W1, degA, degB)

    acc_lo, acc_hi = _scatter_h(src3d_16, dst3d_16, s_lo, s_hi, zer128)

    W2p = jnp.pad(W2, ((0, 0), (0, HALF - NCLS)))
    s2 = pl.pallas_call(
        _mm2_body,
        grid=grid,
        in_specs=[_row_spec(HALF), _row_spec(HALF),
                  _row_spec(DEGW), _row_spec(DEGW),
                  _full_spec(1, HID), _full_spec(HID, HALF)],
        out_specs=_row_spec(HALF),
        out_shape=jax.ShapeDtypeStruct((NPAD, HALF), jnp.float32),
    )(acc_lo, acc_hi, degA, degB, b1.reshape(1, HID), W2p)

    acc2A, acc2B = _scatter_o(src3d_32, dst3d_32, s2, s2, zer128)

    out = pl.pallas_call(
        _out_body,
        grid=grid,
        in_specs=[_row_spec(HALF), _row_spec(HALF),
                  _row_spec(DEGW), _row_spec(DEGW),
                  _full_spec(1, NCLS)],
        out_specs=_row_spec(NCLS),
        out_shape=jax.ShapeDtypeStruct((N, NCLS), jnp.float32),
    )(acc2A, acc2B, degA, degB, b2.reshape(1, NCLS))
    return out


# speedup vs baseline: 18.3987x; 1.0098x over previous
import functools

import jax
import jax.numpy as jnp
from jax import lax
from jax.experimental import pallas as pl
from jax.experimental.pallas import tpu as pltpu
from jax.experimental.pallas import tpu_sc as plsc

N = 10000
E = 320000
D_IN = 128
HID = 256
HALF = 128
NCLS = 64

C = 128
NT = 16
NSC = 2
NPAD = 10240
ROWS_PT = NPAD // NT
DEGW = 128
E_PAD = 327680
BUF_CH = 40

_mesh = plsc.VectorSubcoreMesh(core_axis_name="c", subcore_axis_name="s")


_DEG_NCH = E_PAD // (NSC * NT * C)


@functools.partial(
    pl.kernel,
    out_type=jax.ShapeDtypeStruct((NSC * NPAD, DEGW), jnp.float32),
    mesh=_mesh,
    scratch_types=[
        pltpu.VMEM_SHARED((NPAD, DEGW), jnp.float32),
        pltpu.VMEM((_DEG_NCH, C), jnp.int32),
        pltpu.VMEM((C, DEGW), jnp.float32),
        pltpu.SemaphoreType.DMA,
    ],
)
def _deg_kernel(dst3d, ones, zeros, deg_out, deg_sh, dstb, ones_v, sem):
    cid = lax.axis_index("c")
    sid = lax.axis_index("s")
    wid = cid * NT + sid
    rs = pl.ds(sid * ROWS_PT, ROWS_PT)
    pltpu.sync_copy(zeros, deg_sh.at[rs])
    pltpu.sync_copy(ones, ones_v)
    pltpu.sync_copy(dst3d.at[wid], dstb)
    plsc.subcore_barrier()

    G = 8

    def body(g, carry):
        for t in range(G):
            pltpu.async_copy(ones_v, deg_sh.at[dstb.at[g * G + t]], sem,
                             add=True)
        for t in range(G):
            pltpu.make_async_copy(ones_v, deg_sh.at[dstb.at[g * G + t]],
                                  sem).wait()
        return carry

    lax.fori_loop(0, _DEG_NCH // G, body, 0)
    plsc.subcore_barrier()
    pltpu.sync_copy(deg_sh.at[rs],
                    deg_out.at[pl.ds(cid * NPAD + sid * ROWS_PT, ROWS_PT)])


def _make_scatter(width, split_edges, acc_rows=NPAD):
    nch = E_PAD // (NT * C) // (NSC if split_edges else 1)
    rounds = nch // BUF_CH if nch > BUF_CH else 1
    buf_ch = nch // rounds
    npairs = buf_ch // 2
    rows_pt = acc_rows // NT

    @functools.partial(
        pl.kernel,
        out_type=[jax.ShapeDtypeStruct((acc_rows, width), jnp.float32),
                  jax.ShapeDtypeStruct((acc_rows, width), jnp.float32)],
        mesh=_mesh,
        scratch_types=[
            pltpu.VMEM_SHARED((acc_rows, width), jnp.float32),
            pltpu.VMEM((buf_ch, C), jnp.int32),
            pltpu.VMEM((buf_ch, C), jnp.int32),
            pltpu.VMEM((C, width), jnp.float32),
            pltpu.VMEM((C, width), jnp.float32),
            pltpu.SemaphoreType.DMA,
            pltpu.SemaphoreType.DMA,
        ],
    )
    def k(src3d, dst3d, tabA, tabB, zeros, outA, outB,
          acc_sh, srcb, dstb, rv0, rv1, sem0, sem1):
        cid = lax.axis_index("c")
        sid = lax.axis_index("s")
        if split_edges:
            tile0 = cid * NT + sid
        else:
            tile0 = sid
        rs = pl.ds(sid * rows_pt, rows_pt)
        pltpu.sync_copy(zeros, acc_sh.at[rs])
        plsc.subcore_barrier()

        def run(tab):
            for r in range(rounds):
                pltpu.sync_copy(src3d.at[tile0, pl.ds(r * buf_ch, buf_ch)], srcb)
                pltpu.sync_copy(dst3d.at[tile0, pl.ds(r * buf_ch, buf_ch)], dstb)
                pltpu.async_copy(tab.at[srcb.at[0]], rv0, sem0)

                def body(k_, carry):
                    j0 = 2 * k_
                    pltpu.async_copy(tab.at[srcb.at[j0 + 1]], rv1, sem1)
                    pltpu.make_async_copy(tab.at[srcb.at[j0]], rv0, sem0).wait()
                    pltpu.sync_copy(rv0, acc_sh.at[dstb.at[j0]], add=True)

                    @pl.when(k_ < npairs - 1)
                    def _():
                        pltpu.async_copy(tab.at[srcb.at[j0 + 2]], rv0, sem0)

                    pltpu.make_async_copy(tab.at[srcb.at[j0 + 1]], rv1, sem1).wait()
                    pltpu.sync_copy(rv1, acc_sh.at[dstb.at[j0 + 1]], add=True)
                    return carry

                lax.fori_loop(0, npairs, body, 0)

        @pl.when(cid == 0)
        def _():
            run(tabA)

        @pl.when(cid == 1)
        def _():
            run(tabB)

        plsc.subcore_barrier()

        @pl.when(cid == 0)
        def _():
            pltpu.sync_copy(acc_sh.at[rs], outA.at[rs])

        @pl.when(cid == 1)
        def _():
            pltpu.sync_copy(acc_sh.at[rs], outB.at[rs])

    return k


_scatter_h = _make_scatter(HALF, split_edges=False)
_scatter_o = _make_scatter(HALF, split_edges=True, acc_rows=NPAD // 2)


def _mm1_body(x_ref, w_ref, dA, dB, lo_ref, hi_ref, isd_ref):
    deg = dA[...][:, 0] + dB[...][:, 0]
    isd = lax.rsqrt(jnp.maximum(deg, 1.0))
    isd_ref[...] = jnp.broadcast_to(isd[:, None], isd_ref.shape)
    s = jnp.dot(x_ref[...] * isd[:, None], w_ref[...],
                preferred_element_type=jnp.float32,
                precision=lax.Precision.HIGHEST)
    lo_ref[...] = s[:, :HALF]
    hi_ref[...] = s[:, HALF:]


def _mm2_body(lo_ref, hi_ref, isd8, b1_ref, w2_ref, out_ref):
    isd = isd8[...][:, :1]
    b = b1_ref[...]
    h_lo = jnp.maximum(lo_ref[...] * isd + b[:, :HALF], 0.0) * isd
    h_hi = jnp.maximum(hi_ref[...] * isd + b[:, HALF:], 0.0) * isd
    w2 = w2_ref[...]
    s2 = (jnp.dot(h_lo, w2[:HALF], preferred_element_type=jnp.float32,
                  precision=lax.Precision.HIGHEST)
          + jnp.dot(h_hi, w2[HALF:], preferred_element_type=jnp.float32,
                    precision=lax.Precision.HIGHEST))
    z = jnp.zeros_like(s2)
    top = jnp.concatenate([s2, z], axis=1)
    bot = jnp.concatenate([z, s2], axis=1)
    out_ref[...] = jnp.stack([top, bot], axis=1).reshape(2 * s2.shape[0], HALF)


def _ls64(t):
    m = jnp.max(t, axis=1, keepdims=True)
    y = t - m
    return y - jnp.log(jnp.sum(jnp.exp(y), axis=1, keepdims=True))


def _out_body(aA, aB, ipk_ref, b2_ref, o_ref):
    o = (aA[...] + aB[...]) * ipk_ref[...] + b2_ref[...]
    o_ref[...] = jnp.concatenate([_ls64(o[:, :NCLS]), _ls64(o[:, NCLS:])],
                                 axis=1)


_R = 2048


def _row_spec(w):
    return pl.BlockSpec((_R, w), lambda i: (i, 0))


def _full_spec(a, b):
    return pl.BlockSpec((a, b), lambda i: (0, 0))


def kernel(x, adj, W1, b1, W2, b2):
    src = adj[0].astype(jnp.int32)
    dst = adj[1].astype(jnp.int32)
    pad = N + jnp.arange(E_PAD - E, dtype=jnp.int32) % (NPAD - N)
    src_p = jnp.concatenate([src, pad])
    dst_p = jnp.concatenate([dst, pad])
    src3d_16 = src_p.reshape(NT, E_PAD // (NT * C), C)
    dst3d_16 = dst_p.reshape(NT, E_PAD // (NT * C), C)
    src3d_32 = src_p.reshape(NSC * NT, _DEG_NCH, C)
    dst3d_32 = dst_p.reshape(NSC * NT, _DEG_NCH, C)
    ones128 = jnp.ones((C, DEGW), jnp.float32)
    zer128 = jnp.zeros((ROWS_PT, HALF), jnp.float32)

    deg2 = _deg_kernel(dst3d_32, ones128, zer128)

    grid = (NPAD // _R,)
    s_lo, s_hi, isd8 = pl.pallas_call(
        _mm1_body,
        grid=grid,
        in_specs=[_row_spec(D_IN), _full_spec(D_IN, HID),
                  pl.BlockSpec((_R, DEGW), lambda i: (i, 0)),
                  pl.BlockSpec((_R, DEGW), lambda i: (i + NPAD // _R, 0))],
        out_specs=[_row_spec(HALF), _row_spec(HALF), _row_spec(8)],
        out_shape=[jax.ShapeDtypeStruct((NPAD, HALF), jnp.float32),
                   jax.ShapeDtypeStruct((NPAD, HALF), jnp.float32),
                   jax.ShapeDtypeStruct((NPAD, 8), jnp.float32)],
    )(x, W1, deg2, deg2)

    acc_lo, acc_hi = _scatter_h(src3d_16, dst3d_16, s_lo, s_hi, zer128)

    s2d = pl.pallas_call(
        _mm2_body,
        grid=grid,
        in_specs=[_row_spec(HALF), _row_spec(HALF), _row_spec(8),
                  _full_spec(1, HID), _full_spec(HID, NCLS)],
        out_specs=pl.BlockSpec((2 * _R, HALF), lambda i: (i, 0)),
        out_shape=jax.ShapeDtypeStruct((2 * NPAD, HALF), jnp.float32),
    )(acc_lo, acc_hi, isd8, b1.reshape(1, HID), W2)

    srcD = (src_p * 2 + (dst_p & 1)).reshape(NSC * NT, _DEG_NCH, C)
    dstD = (dst_p >> 1).reshape(NSC * NT, _DEG_NCH, C)
    zer320 = jnp.zeros((NPAD // 2 // NT, HALF), jnp.float32)
    acc2A, acc2B = _scatter_o(srcD, dstD, s2d, s2d, zer320)

    isd_pk = jnp.repeat(isd8[:, 0], NCLS).reshape(NPAD // 2, HALF)
    b2pk = jnp.concatenate([b2, b2]).reshape(1, HALF)
    out_pk = pl.pallas_call(
        _out_body,
        grid=grid,
        in_specs=[pl.BlockSpec((_R // 2, HALF), lambda i: (i, 0)),
                  pl.BlockSpec((_R // 2, HALF), lambda i: (i, 0)),
                  pl.BlockSpec((_R // 2, HALF), lambda i: (i, 0)),
                  _full_spec(1, HALF)],
        out_specs=pl.BlockSpec((_R // 2, HALF), lambda i: (i, 0)),
        out_shape=jax.ShapeDtypeStruct((N // 2, HALF), jnp.float32),
    )(acc2A, acc2B, isd_pk, b2pk)
    return out_pk.reshape(N, NCLS)
